# Initial kernel scaffold; baseline (speedup 1.0000x reference)
import jax
import jax.numpy as jnp
from jax.experimental import pallas as pl


def kernel(x, W):
    raise NotImplementedError("write your pallas kernel here")


# speedup vs baseline: 3.1834x; 3.1834x over previous
import functools

import jax
import jax.numpy as jnp
from jax import lax
from jax.experimental import pallas as pl
from jax.experimental.pallas import tpu as pltpu
from jax.experimental.pallas import tpu_sc as plsc

VOCAB = 100000
DIM = 64
BATCH = 4096
HIST = 200

B = BATCH * HIST
NC, NS = 2, 16
NW = NC * NS
BPW = B // NW
CHUNK = 128
NCH = BPW // CHUNK

_mesh = plsc.VectorSubcoreMesh(core_axis_name="c", subcore_axis_name="s")


@functools.partial(
    pl.kernel,
    mesh=_mesh,
    out_type=jax.ShapeDtypeStruct((B, DIM), jnp.float32),
    scratch_types=[
        pltpu.VMEM((CHUNK,), jnp.int32),
        pltpu.VMEM((CHUNK, DIM), jnp.float32),
        pltpu.SemaphoreType.DMA,
    ],
    compiler_params=pltpu.CompilerParams(use_tc_tiling_on_sc=False),
)
def _gather_kernel(table_hbm, idx_hbm, out_hbm, idx_v, rows_v, sem):
    wid = lax.axis_index("s") * NC + lax.axis_index("c")
    base = wid * BPW

    def body(g, carry):
        off = base + g * CHUNK
        pltpu.sync_copy(idx_hbm.at[pl.ds(off, CHUNK)], idx_v)
        pltpu.async_copy(table_hbm.at[idx_v], rows_v, sem).wait()
        pltpu.sync_copy(rows_v, out_hbm.at[pl.ds(off, CHUNK)])
        return carry

    lax.fori_loop(0, NCH, body, 0)


def kernel(x, W):
    flat = x.reshape(B).astype(jnp.int32)
    out = _gather_kernel(W, flat)
    return out.reshape(BATCH, HIST, DIM)


# speedup vs baseline: 4.2474x; 1.3342x over previous
import functools

import jax
import jax.numpy as jnp
from jax import lax
from jax.experimental import pallas as pl
from jax.experimental.pallas import tpu as pltpu
from jax.experimental.pallas import tpu_sc as plsc

VOCAB = 100000
DIM = 64
BATCH = 4096
HIST = 200

B = BATCH * HIST
NC, NS = 2, 16
NW = NC * NS
BPW = B // NW
CHUNK = 128
K = 5
GROUP = K * CHUNK
NG = BPW // GROUP
NB = 2
NITER = NG // NB

_mesh = plsc.VectorSubcoreMesh(core_axis_name="c", subcore_axis_name="s")


@functools.partial(
    pl.kernel,
    mesh=_mesh,
    out_type=jax.ShapeDtypeStruct((B, DIM), jnp.float32),
    scratch_types=[
        pltpu.VMEM((NB, GROUP), jnp.int32),
        pltpu.VMEM((NB, GROUP, DIM), jnp.float32),
        pltpu.SemaphoreType.DMA,
        pltpu.SemaphoreType.DMA,
        pltpu.SemaphoreType.DMA,
        pltpu.SemaphoreType.DMA,
        pltpu.SemaphoreType.DMA,
        pltpu.SemaphoreType.DMA,
    ],
    compiler_params=pltpu.CompilerParams(use_tc_tiling_on_sc=False),
)
def _gather_kernel(table_hbm, idx_hbm, out_hbm, idx_v, rows_v,
                   sidx0, sidx1, sg0, sg1, sw0, sw1):
    sidx = (sidx0, sidx1)
    sg = (sg0, sg1)
    sw = (sw0, sw1)
    wid = lax.axis_index("s") * NC + lax.axis_index("c")
    base = wid * BPW

    def idx_fetch(slot, g):
        pltpu.async_copy(idx_hbm.at[pl.ds(base + g * GROUP, GROUP)],
                         idx_v.at[slot], sidx[slot])

    for b in range(NB):
        idx_fetch(b, b)

    def body(i, carry):
        for b in range(NB):
            g = i * NB + b
            roff = base + g * GROUP

            @pl.when(i > 0)
            def _drain_write():
                pltpu.make_async_copy(
                    rows_v.at[b], out_hbm.at[pl.ds(roff, GROUP)],
                    sw[b]).wait()

            pltpu.make_async_copy(
                idx_hbm.at[pl.ds(base, GROUP)], idx_v.at[b], sidx[b]).wait()

            for j in range(K):
                pltpu.async_copy(
                    table_hbm.at[idx_v.at[b].at[pl.ds(j * CHUNK, CHUNK)]],
                    rows_v.at[b].at[pl.ds(j * CHUNK, CHUNK)], sg[b])
            for j in range(K):
                pltpu.make_async_copy(
                    table_hbm.at[idx_v.at[b].at[pl.ds(j * CHUNK, CHUNK)]],
                    rows_v.at[b].at[pl.ds(j * CHUNK, CHUNK)], sg[b]).wait()

            @pl.when(i < NITER - 1)
            def _prefetch():
                idx_fetch(b, g + NB)

            pltpu.async_copy(
                rows_v.at[b], out_hbm.at[pl.ds(roff, GROUP)], sw[b])
        return carry

    lax.fori_loop(0, NITER, body, 0)

    for b in range(NB):
        pltpu.make_async_copy(
            rows_v.at[b], out_hbm.at[pl.ds(base, GROUP)], sw[b]).wait()


def kernel(x, W):
    flat = x.reshape(B).astype(jnp.int32)
    out = _gather_kernel(W, flat)
    return out.reshape(BATCH, HIST, DIM)
